# Initial kernel scaffold; baseline (speedup 1.0000x reference)
import jax
import jax.numpy as jnp
from jax.experimental import pallas as pl


def kernel(features, edge_index, etypes, Wb1, wc1, loopW1, b1, Wb2, wc2, loopW2, b2):
    raise NotImplementedError("write your pallas kernel here")


# speedup vs baseline: 19.9340x; 19.9340x over previous
import functools

import jax
import jax.numpy as jnp
from jax import lax
from jax.experimental import pallas as pl
from jax.experimental.pallas import tpu as pltpu
from jax.experimental.pallas import tpu_sc as plsc

_NC = 2
_NS = 16
_NW = _NC * _NS

_CHUNK = 80


def _gidx_body(n, et_ref, src_ref, o_ref):
    o_ref[...] = et_ref[...] * n + src_ref[...]


def _flat_gather_index(etypes, src, n):
    e = etypes.shape[0]
    cols = 512
    rows = e // cols
    et2 = etypes.reshape(rows, cols)
    src2 = src.reshape(rows, cols)
    out = pl.pallas_call(
        functools.partial(_gidx_body, n),
        out_shape=jax.ShapeDtypeStruct((rows, cols), jnp.int32),
    )(et2, src2)
    return out.reshape(e)


def _compose_body(wc_ref, wb_ref, o_ref):
    o_ref[...] = jnp.dot(wc_ref[...], wb_ref[...],
                         preferred_element_type=jnp.float32)


def _compose_w(wc, wb):
    b, d, _ = wb.shape
    r = wc.shape[0]
    wb_flat = wb.reshape(b, d * d)
    cols = 2048
    grid = (d * d // cols,)
    out = pl.pallas_call(
        _compose_body,
        grid=grid,
        in_specs=[
            pl.BlockSpec((r, b), lambda i: (0, 0)),
            pl.BlockSpec((b, cols), lambda i: (0, i)),
        ],
        out_specs=pl.BlockSpec((r, cols), lambda i: (0, i)),
        out_shape=jax.ShapeDtypeStruct((r, d * d), jnp.float32),
    )(wc, wb_flat)
    return out.reshape(r, d, d)


def _transform_body(h_ref, w_ref, o_ref):
    o_ref[...] = jnp.dot(h_ref[...], w_ref[0],
                         preferred_element_type=jnp.float32)


def _transform(h, w, nbk):
    n, d = h.shape
    r = w.shape[0]
    nb = n // nbk
    out = pl.pallas_call(
        _transform_body,
        grid=(nb, r),
        in_specs=[
            pl.BlockSpec((nbk, d), lambda i, j: (i, 0)),
            pl.BlockSpec((1, d, d), lambda i, j: (j, 0, 0)),
        ],
        out_specs=pl.BlockSpec((nbk, d), lambda i, j: (j * nb + i, 0)),
        out_shape=jax.ShapeDtypeStruct((r * n, d), jnp.float32),
    )(h, w)
    return out


def _combine_body(part_ref, h_ref, lw_ref, b_ref, o_ref):
    loop = jnp.dot(h_ref[...], lw_ref[...], preferred_element_type=jnp.float32)
    o_ref[...] = jnp.maximum(part_ref[0] + part_ref[1] + loop + b_ref[...], 0.0)


def _combine(part, h, loop_w, bias2d, nbk):
    n, d = h.shape
    nb = n // nbk
    return pl.pallas_call(
        _combine_body,
        grid=(nb,),
        in_specs=[
            pl.BlockSpec((2, nbk, d), lambda i: (0, i, 0)),
            pl.BlockSpec((nbk, d), lambda i: (i, 0)),
            pl.BlockSpec((d, d), lambda i: (0, 0)),
            pl.BlockSpec((1, d), lambda i: (0, 0)),
        ],
        out_specs=pl.BlockSpec((nbk, d), lambda i: (i, 0)),
        out_shape=jax.ShapeDtypeStruct((n, d), jnp.float32),
    )(part, h, loop_w, bias2d)


def _make_edge_agg(n, d, e):
    epw = e // _NW
    nchunk = epw // _CHUNK
    npt = (n // _NS) // 8 * 8
    rem = n - _NS * npt

    mesh = plsc.VectorSubcoreMesh(core_axis_name="c", subcore_axis_name="s")

    @functools.partial(
        pl.kernel,
        out_type=jax.ShapeDtypeStruct((_NC, n, d), jnp.float32),
        mesh=mesh,
        scratch_types=[
            pltpu.VMEM((nchunk, _CHUNK), jnp.int32),
            pltpu.VMEM((nchunk, _CHUNK), jnp.int32),
            pltpu.VMEM((_CHUNK, d), jnp.float32),
            pltpu.VMEM((_CHUNK, d), jnp.float32),
            pltpu.VMEM_SHARED((n, d), jnp.float32),
            pltpu.SemaphoreType.DMA,
            pltpu.SemaphoreType.DMA,
        ],
    )
    def edge_agg(hw_hbm, gidx_hbm, dst_hbm, zeros_hbm, out_hbm,
                 idx_v, dst_v, rows_a, rows_b, acc_v, sem_a, sem_b):
        c = lax.axis_index("c")
        s = lax.axis_index("s")
        wid = s * _NC + c

        pltpu.sync_copy(gidx_hbm.at[wid], idx_v)
        pltpu.sync_copy(dst_hbm.at[wid], dst_v)
        pltpu.sync_copy(zeros_hbm.at[pl.ds(s * npt, npt)],
                        acc_v.at[pl.ds(s * npt, npt)])
        if rem:
            @pl.when(s == _NS - 1)
            def _():
                pltpu.sync_copy(zeros_hbm.at[pl.ds(_NS * npt, rem)],
                                acc_v.at[pl.ds(_NS * npt, rem)])
        plsc.subcore_barrier()

        def body(j, carry):
            pltpu.async_copy(hw_hbm.at[idx_v.at[j]], rows_a, sem_a).wait()
            pltpu.sync_copy(rows_a, acc_v.at[dst_v.at[j]], add=True)
            return carry

        lax.fori_loop(0, nchunk, body, 0)

        plsc.subcore_barrier()
        pltpu.sync_copy(acc_v.at[pl.ds(s * npt, npt)],
                        out_hbm.at[c, pl.ds(s * npt, npt)])
        if rem:
            @pl.when(s == _NS - 1)
            def _():
                pltpu.sync_copy(acc_v.at[pl.ds(_NS * npt, rem)],
                                out_hbm.at[c, pl.ds(_NS * npt, rem)])

    return edge_agg


def _layer(h, gidx3, dst3, zeros_nd, wb, wc, loop_w, bias2d, edge_agg, nbk):
    w = _compose_w(wc, wb)
    hw = _transform(h, w, nbk)
    part = edge_agg(hw, gidx3, dst3, zeros_nd)
    return _combine(part, h, loop_w, bias2d, nbk)


def kernel(features, edge_index, etypes, Wb1, wc1, loopW1, b1,
           Wb2, wc2, loopW2, b2):
    n, d = features.shape
    e = etypes.shape[0]
    src = edge_index[0]
    dst = edge_index[1]

    gidx = _flat_gather_index(etypes, src, n)
    epw = e // _NW
    nchunk = epw // _CHUNK
    gidx3 = gidx.reshape(_NW, nchunk, _CHUNK)
    dst3 = dst.reshape(_NW, nchunk, _CHUNK)
    zeros_nd = jnp.zeros((n, d), jnp.float32)

    edge_agg = _make_edge_agg(n, d, e)
    nbk = 1000

    h1 = _layer(features, gidx3, dst3, zeros_nd, Wb1, wc1, loopW1,
                b1.reshape(1, d), edge_agg, nbk)
    h2 = _layer(h1, gidx3, dst3, zeros_nd, Wb2, wc2, loopW2,
                b2.reshape(1, d), edge_agg, nbk)
    return h2


# speedup vs baseline: 27.0119x; 1.3551x over previous
import functools

import jax
import jax.numpy as jnp
from jax import lax
from jax.experimental import pallas as pl
from jax.experimental.pallas import tpu as pltpu
from jax.experimental.pallas import tpu_sc as plsc

_NC = 2
_NS = 16
_NW = _NC * _NS

_CHUNK = 80


def _gidx_body(n, et_ref, src_ref, o_ref):
    o_ref[...] = et_ref[...] * n + src_ref[...]


def _flat_gather_index(etypes, src, n):
    e = etypes.shape[0]
    cols = 512
    rows = e // cols
    et2 = etypes.reshape(rows, cols)
    src2 = src.reshape(rows, cols)
    out = pl.pallas_call(
        functools.partial(_gidx_body, n),
        out_shape=jax.ShapeDtypeStruct((rows, cols), jnp.int32),
    )(et2, src2)
    return out.reshape(e)


def _compose_body(wc_ref, wb_ref, o_ref):
    o_ref[...] = jnp.dot(wc_ref[...], wb_ref[...],
                         preferred_element_type=jnp.float32)


def _compose_w(wc, wb):
    b, d, _ = wb.shape
    r = wc.shape[0]
    wb_flat = wb.reshape(b, d * d)
    cols = 2048
    grid = (d * d // cols,)
    out = pl.pallas_call(
        _compose_body,
        grid=grid,
        in_specs=[
            pl.BlockSpec((r, b), lambda i: (0, 0)),
            pl.BlockSpec((b, cols), lambda i: (0, i)),
        ],
        out_specs=pl.BlockSpec((r, cols), lambda i: (0, i)),
        out_shape=jax.ShapeDtypeStruct((r, d * d), jnp.float32),
    )(wc, wb_flat)
    return out.reshape(r, d, d)


def _transform_body(h_ref, w_ref, o_ref):
    o_ref[...] = jnp.dot(h_ref[...], w_ref[0],
                         preferred_element_type=jnp.float32)


def _transform(h, w, nbk):
    n, d = h.shape
    r = w.shape[0]
    nb = n // nbk
    out = pl.pallas_call(
        _transform_body,
        grid=(nb, r),
        in_specs=[
            pl.BlockSpec((nbk, d), lambda i, j: (i, 0)),
            pl.BlockSpec((1, d, d), lambda i, j: (j, 0, 0)),
        ],
        out_specs=pl.BlockSpec((nbk, d), lambda i, j: (j * nb + i, 0)),
        out_shape=jax.ShapeDtypeStruct((r * n, d), jnp.float32),
    )(h, w)
    return out


def _combine_body(part_ref, h_ref, lw_ref, b_ref, o_ref):
    loop = jnp.dot(h_ref[...], lw_ref[...], preferred_element_type=jnp.float32)
    o_ref[...] = jnp.maximum(part_ref[0] + part_ref[1] + loop + b_ref[...], 0.0)


def _combine(part, h, loop_w, bias2d, nbk):
    n, d = h.shape
    nb = n // nbk
    return pl.pallas_call(
        _combine_body,
        grid=(nb,),
        in_specs=[
            pl.BlockSpec((2, nbk, d), lambda i: (0, i, 0)),
            pl.BlockSpec((nbk, d), lambda i: (i, 0)),
            pl.BlockSpec((d, d), lambda i: (0, 0)),
            pl.BlockSpec((1, d), lambda i: (0, 0)),
        ],
        out_specs=pl.BlockSpec((nbk, d), lambda i: (i, 0)),
        out_shape=jax.ShapeDtypeStruct((n, d), jnp.float32),
    )(part, h, loop_w, bias2d)


def _make_edge_agg(n, d, e):
    epw = e // _NW
    nchunk = epw // _CHUNK
    nblk = 5
    cpb = nchunk // nblk
    npt = (n // _NS) // 8 * 8
    rem = n - _NS * npt

    mesh = plsc.VectorSubcoreMesh(core_axis_name="c", subcore_axis_name="s")

    @functools.partial(
        pl.kernel,
        out_type=jax.ShapeDtypeStruct((_NC, n, d), jnp.float32),
        mesh=mesh,
        scratch_types=[
            pltpu.VMEM((cpb, _CHUNK), jnp.int32),
            pltpu.VMEM((cpb, _CHUNK), jnp.int32),
            pltpu.VMEM((_CHUNK, d), jnp.float32),
            pltpu.VMEM((_CHUNK, d), jnp.float32),
            pltpu.VMEM_SHARED((n, d), jnp.float32),
            pltpu.SemaphoreType.DMA,
            pltpu.SemaphoreType.DMA,
        ],
    )
    def edge_agg(hw_hbm, gidx_hbm, dst_hbm, zeros_hbm, out_hbm,
                 idx_v, dst_v, rows_a, rows_b, acc_v, sem_a, sem_b):
        c = lax.axis_index("c")
        s = lax.axis_index("s")
        wid = s * _NC + c

        pltpu.sync_copy(zeros_hbm.at[pl.ds(s * npt, npt)],
                        acc_v.at[pl.ds(s * npt, npt)])
        if rem:
            @pl.when(s == _NS - 1)
            def _():
                pltpu.sync_copy(zeros_hbm.at[pl.ds(_NS * npt, rem)],
                                acc_v.at[pl.ds(_NS * npt, rem)])
        plsc.subcore_barrier()

        def block(k, carry):
            pltpu.sync_copy(gidx_hbm.at[wid, k], idx_v)
            pltpu.sync_copy(dst_hbm.at[wid, k], dst_v)
            pltpu.async_copy(hw_hbm.at[idx_v.at[0]], rows_a, sem_a)

            def body(i, c2):
                j = 2 * i
                pltpu.async_copy(hw_hbm.at[idx_v.at[j + 1]], rows_b, sem_b)
                pltpu.make_async_copy(hw_hbm.at[idx_v.at[j]], rows_a,
                                      sem_a).wait()
                pltpu.sync_copy(rows_a, acc_v.at[dst_v.at[j]], add=True)

                @pl.when(j + 2 < cpb)
                def _():
                    pltpu.async_copy(hw_hbm.at[idx_v.at[j + 2]], rows_a, sem_a)

                pltpu.make_async_copy(hw_hbm.at[idx_v.at[j + 1]], rows_b,
                                      sem_b).wait()
                pltpu.sync_copy(rows_b, acc_v.at[dst_v.at[j + 1]], add=True)
                return c2

            lax.fori_loop(0, cpb // 2, body, 0)
            if cpb % 2:
                j_last = cpb - 1
                pltpu.make_async_copy(hw_hbm.at[idx_v.at[j_last]], rows_a,
                                      sem_a).wait()
                pltpu.sync_copy(rows_a, acc_v.at[dst_v.at[j_last]], add=True)
            return carry

        lax.fori_loop(0, nblk, block, 0)

        plsc.subcore_barrier()
        pltpu.sync_copy(acc_v.at[pl.ds(s * npt, npt)],
                        out_hbm.at[c, pl.ds(s * npt, npt)])
        if rem:
            @pl.when(s == _NS - 1)
            def _():
                pltpu.sync_copy(acc_v.at[pl.ds(_NS * npt, rem)],
                                out_hbm.at[c, pl.ds(_NS * npt, rem)])

    return edge_agg


def _layer(h, gidx3, dst3, zeros_nd, wb, wc, loop_w, bias2d, edge_agg, nbk):
    w = _compose_w(wc, wb)
    hw = _transform(h, w, nbk)
    part = edge_agg(hw, gidx3, dst3, zeros_nd)
    return _combine(part, h, loop_w, bias2d, nbk)


def kernel(features, edge_index, etypes, Wb1, wc1, loopW1, b1,
           Wb2, wc2, loopW2, b2):
    n, d = features.shape
    e = etypes.shape[0]
    src = edge_index[0]
    dst = edge_index[1]

    gidx = _flat_gather_index(etypes, src, n)
    epw = e // _NW
    nchunk = epw // _CHUNK
    nblk = 5
    gidx3 = gidx.reshape(_NW, nblk, nchunk // nblk, _CHUNK)
    dst3 = dst.reshape(_NW, nblk, nchunk // nblk, _CHUNK)
    zeros_nd = jnp.zeros((n, d), jnp.float32)

    edge_agg = _make_edge_agg(n, d, e)
    nbk = 1000

    h1 = _layer(features, gidx3, dst3, zeros_nd, Wb1, wc1, loopW1,
                b1.reshape(1, d), edge_agg, nbk)
    h2 = _layer(h1, gidx3, dst3, zeros_nd, Wb2, wc2, loopW2,
                b2.reshape(1, d), edge_agg, nbk)
    return h2


# speedup vs baseline: 30.5919x; 1.1325x over previous
import functools

import jax
import jax.numpy as jnp
from jax import lax
from jax.experimental import pallas as pl
from jax.experimental.pallas import tpu as pltpu
from jax.experimental.pallas import tpu_sc as plsc

_NC = 2
_NS = 16
_NW = _NC * _NS

_CHUNK = 80


def _gidx_body(n, et_ref, src_ref, o_ref):
    o_ref[...] = et_ref[...] * n + src_ref[...]


def _flat_gather_index(etypes, src, n):
    e = etypes.shape[0]
    cols = 512
    rows = e // cols
    et2 = etypes.reshape(rows, cols)
    src2 = src.reshape(rows, cols)
    out = pl.pallas_call(
        functools.partial(_gidx_body, n),
        out_shape=jax.ShapeDtypeStruct((rows, cols), jnp.int32),
    )(et2, src2)
    return out.reshape(e)


def _compose_body(wc_ref, wb_ref, o_ref):
    o_ref[...] = jnp.dot(wc_ref[...], wb_ref[...],
                         preferred_element_type=jnp.float32)


def _compose_w(wc, wb):
    b, d, _ = wb.shape
    r = wc.shape[0]
    wb_flat = wb.reshape(b, d * d)
    cols = 2048
    grid = (d * d // cols,)
    out = pl.pallas_call(
        _compose_body,
        grid=grid,
        in_specs=[
            pl.BlockSpec((r, b), lambda i: (0, 0)),
            pl.BlockSpec((b, cols), lambda i: (0, i)),
        ],
        out_specs=pl.BlockSpec((r, cols), lambda i: (0, i)),
        out_shape=jax.ShapeDtypeStruct((r, d * d), jnp.float32),
    )(wc, wb_flat)
    return out.reshape(r, d, d)


def _transform_body(h_ref, w_ref, o_ref):
    o_ref[...] = jnp.dot(h_ref[...], w_ref[0],
                         preferred_element_type=jnp.float32)


def _transform(h, w, nbk):
    n, d = h.shape
    r = w.shape[0]
    nb = n // nbk
    out = pl.pallas_call(
        _transform_body,
        grid=(nb, r),
        in_specs=[
            pl.BlockSpec((nbk, d), lambda i, j: (i, 0)),
            pl.BlockSpec((1, d, d), lambda i, j: (j, 0, 0)),
        ],
        out_specs=pl.BlockSpec((nbk, d), lambda i, j: (j * nb + i, 0)),
        out_shape=jax.ShapeDtypeStruct((r * n, d), jnp.float32),
    )(h, w)
    return out


def _combine_body(part_ref, h_ref, lw_ref, b_ref, o_ref):
    loop = jnp.dot(h_ref[...], lw_ref[...], preferred_element_type=jnp.float32)
    o_ref[...] = jnp.maximum(part_ref[0] + part_ref[1] + loop + b_ref[...], 0.0)


def _combine(part, h, loop_w, bias2d, nbk):
    n, d = h.shape
    nb = n // nbk
    return pl.pallas_call(
        _combine_body,
        grid=(nb,),
        in_specs=[
            pl.BlockSpec((2, nbk, d), lambda i: (0, i, 0)),
            pl.BlockSpec((nbk, d), lambda i: (i, 0)),
            pl.BlockSpec((d, d), lambda i: (0, 0)),
            pl.BlockSpec((1, d), lambda i: (0, 0)),
        ],
        out_specs=pl.BlockSpec((nbk, d), lambda i: (i, 0)),
        out_shape=jax.ShapeDtypeStruct((n, d), jnp.float32),
    )(part, h, loop_w, bias2d)


def _make_edge_agg(n, d, e):
    epw = e // _NW
    nchunk = epw // _CHUNK
    nblk = 5
    cpb = nchunk // nblk
    npt = (n // _NS) // 8 * 8
    rem = n - _NS * npt

    mesh = plsc.VectorSubcoreMesh(core_axis_name="c", subcore_axis_name="s")

    @functools.partial(
        pl.kernel,
        out_type=jax.ShapeDtypeStruct((_NC, n, d), jnp.float32),
        mesh=mesh,
        scratch_types=[
            pltpu.VMEM((cpb, _CHUNK), jnp.int32),
            pltpu.VMEM((cpb, _CHUNK), jnp.int32),
            pltpu.VMEM((_CHUNK, d), jnp.float32),
            pltpu.VMEM((_CHUNK, d), jnp.float32),
            pltpu.VMEM_SHARED((n, d), jnp.float32),
            pltpu.SemaphoreType.DMA,
            pltpu.SemaphoreType.DMA,
        ],
    )
    def edge_agg(hw_hbm, gidx_hbm, dst_hbm, zeros_hbm, out_hbm,
                 idx_v, dst_v, rows_a, rows_b, acc_v, sem_a, sem_b):
        c = lax.axis_index("c")
        s = lax.axis_index("s")
        wid = s * _NC + c

        pltpu.sync_copy(zeros_hbm.at[pl.ds(s * npt, npt)],
                        acc_v.at[pl.ds(s * npt, npt)])
        if rem:
            @pl.when(s == _NS - 1)
            def _():
                pltpu.sync_copy(zeros_hbm.at[pl.ds(_NS * npt, rem)],
                                acc_v.at[pl.ds(_NS * npt, rem)])
        plsc.subcore_barrier()

        def block(k, carry):
            pltpu.sync_copy(gidx_hbm.at[wid, k], idx_v)
            pltpu.sync_copy(dst_hbm.at[wid, k], dst_v)
            pltpu.async_copy(hw_hbm.at[idx_v.at[0]], rows_a, sem_a)

            def body(i, c2):
                j = 2 * i
                pltpu.async_copy(hw_hbm.at[idx_v.at[j + 1]], rows_b, sem_b)
                pltpu.make_async_copy(hw_hbm.at[idx_v.at[j]], rows_a,
                                      sem_a).wait()
                pltpu.sync_copy(rows_a, acc_v.at[dst_v.at[j]], add=True)

                @pl.when(j + 2 < cpb)
                def _():
                    pltpu.async_copy(hw_hbm.at[idx_v.at[j + 2]], rows_a, sem_a)

                pltpu.make_async_copy(hw_hbm.at[idx_v.at[j + 1]], rows_b,
                                      sem_b).wait()
                pltpu.sync_copy(rows_b, acc_v.at[dst_v.at[j + 1]], add=True)
                return c2

            lax.fori_loop(0, cpb // 2, body, 0)
            if cpb % 2:
                j_last = cpb - 1
                pltpu.make_async_copy(hw_hbm.at[idx_v.at[j_last]], rows_a,
                                      sem_a).wait()
                pltpu.sync_copy(rows_a, acc_v.at[dst_v.at[j_last]], add=True)
            return carry

        lax.fori_loop(0, nblk, block, 0)

        plsc.subcore_barrier()
        pltpu.sync_copy(acc_v.at[pl.ds(s * npt, npt)],
                        out_hbm.at[c, pl.ds(s * npt, npt)])
        if rem:
            @pl.when(s == _NS - 1)
            def _():
                pltpu.sync_copy(acc_v.at[pl.ds(_NS * npt, rem)],
                                out_hbm.at[c, pl.ds(_NS * npt, rem)])

    return edge_agg


def _layer(h, gidx3, dst3, zeros_nd, wb, wc, loop_w, bias2d, edge_agg, nbk):
    w = _compose_w(wc, wb)
    hw = _transform(h, w, nbk)
    part = edge_agg(hw, gidx3, dst3, zeros_nd)
    return _combine(part, h, loop_w, bias2d, nbk)


def kernel(features, edge_index, etypes, Wb1, wc1, loopW1, b1,
           Wb2, wc2, loopW2, b2):
    n, d = features.shape
    e = etypes.shape[0]
    src = edge_index[0]
    dst = edge_index[1]

    gidx = _flat_gather_index(etypes, src, n)
    epw = e // _NW
    nchunk = epw // _CHUNK
    nblk = 5
    gidx3 = gidx.reshape(_NW, nblk, nchunk // nblk, _CHUNK)
    dst3 = dst.reshape(_NW, nblk, nchunk // nblk, _CHUNK)
    zeros_nd = jnp.zeros((n, d), jnp.float32)

    edge_agg = _make_edge_agg(n, d, e)
    nbk = 2000

    h1 = _layer(features, gidx3, dst3, zeros_nd, Wb1, wc1, loopW1,
                b1.reshape(1, d), edge_agg, nbk)
    h2 = _layer(h1, gidx3, dst3, zeros_nd, Wb2, wc2, loopW2,
                b2.reshape(1, d), edge_agg, nbk)
    return h2


# speedup vs baseline: 32.6654x; 1.0678x over previous
import functools

import jax
import jax.numpy as jnp
from jax import lax
from jax.experimental import pallas as pl
from jax.experimental.pallas import tpu as pltpu
from jax.experimental.pallas import tpu_sc as plsc

_NC = 2
_NS = 16
_NW = _NC * _NS

_CHUNK = 80


def _gidx_body(n, et_ref, src_ref, o_ref):
    o_ref[...] = et_ref[...] * n + src_ref[...]


def _flat_gather_index(etypes, src, n):
    e = etypes.shape[0]
    cols = 512
    rows = e // cols
    et2 = etypes.reshape(rows, cols)
    src2 = src.reshape(rows, cols)
    out = pl.pallas_call(
        functools.partial(_gidx_body, n),
        out_shape=jax.ShapeDtypeStruct((rows, cols), jnp.int32),
    )(et2, src2)
    return out.reshape(e)


def _compose_body(wc_ref, wb_ref, o_ref):
    o_ref[...] = jnp.dot(wc_ref[...], wb_ref[...],
                         preferred_element_type=jnp.float32)


def _compose_w(wc, wb):
    b, d, _ = wb.shape
    r = wc.shape[0]
    wb_flat = wb.reshape(b, d * d)
    cols = 2048
    grid = (d * d // cols,)
    out = pl.pallas_call(
        _compose_body,
        grid=grid,
        in_specs=[
            pl.BlockSpec((r, b), lambda i: (0, 0)),
            pl.BlockSpec((b, cols), lambda i: (0, i)),
        ],
        out_specs=pl.BlockSpec((r, cols), lambda i: (0, i)),
        out_shape=jax.ShapeDtypeStruct((r, d * d), jnp.float32),
    )(wc, wb_flat)
    return out.reshape(r, d, d)


def _transform_body(h_ref, w_ref, o_ref):
    o_ref[...] = jnp.dot(h_ref[...], w_ref[0],
                         preferred_element_type=jnp.float32)


def _transform(h, w, nbk):
    n, d = h.shape
    r = w.shape[0]
    nb = n // nbk
    out = pl.pallas_call(
        _transform_body,
        grid=(nb, r),
        in_specs=[
            pl.BlockSpec((nbk, d), lambda i, j: (i, 0)),
            pl.BlockSpec((1, d, d), lambda i, j: (j, 0, 0)),
        ],
        out_specs=pl.BlockSpec((nbk, d), lambda i, j: (j * nb + i, 0)),
        out_shape=jax.ShapeDtypeStruct((r * n, d), jnp.float32),
    )(h, w)
    return out


def _combine_body(part_ref, h_ref, lw_ref, b_ref, o_ref):
    loop = jnp.dot(h_ref[...], lw_ref[...], preferred_element_type=jnp.float32)
    o_ref[...] = jnp.maximum(part_ref[0] + part_ref[1] + loop + b_ref[...], 0.0)


def _combine(part, h, loop_w, bias2d, nbk):
    n, d = h.shape
    nb = n // nbk
    return pl.pallas_call(
        _combine_body,
        grid=(nb,),
        in_specs=[
            pl.BlockSpec((2, nbk, d), lambda i: (0, i, 0)),
            pl.BlockSpec((nbk, d), lambda i: (i, 0)),
            pl.BlockSpec((d, d), lambda i: (0, 0)),
            pl.BlockSpec((1, d), lambda i: (0, 0)),
        ],
        out_specs=pl.BlockSpec((nbk, d), lambda i: (i, 0)),
        out_shape=jax.ShapeDtypeStruct((n, d), jnp.float32),
    )(part, h, loop_w, bias2d)


def _make_edge_agg(n, d, e):
    epw = e // _NW
    nchunk = epw // _CHUNK
    nblk = 5
    cpb = nchunk // nblk
    npt = (n // _NS) // 8 * 8
    rem = n - _NS * npt

    mesh = plsc.VectorSubcoreMesh(core_axis_name="c", subcore_axis_name="s")

    @functools.partial(
        pl.kernel,
        out_type=jax.ShapeDtypeStruct((_NC, n, d), jnp.float32),
        mesh=mesh,
        scratch_types=[
            pltpu.VMEM((cpb, _CHUNK), jnp.int32),
            pltpu.VMEM((cpb, _CHUNK), jnp.int32),
            pltpu.VMEM((_CHUNK, d), jnp.float32),
            pltpu.VMEM((_CHUNK, d), jnp.float32),
            pltpu.VMEM_SHARED((n, d), jnp.float32),
            pltpu.SemaphoreType.DMA,
            pltpu.SemaphoreType.DMA,
        ],
    )
    def edge_agg(hw_hbm, gidx_hbm, dst_hbm, zeros_hbm, out_hbm,
                 idx_v, dst_v, rows_a, rows_b, acc_v, sem_a, sem_b):
        c = lax.axis_index("c")
        s = lax.axis_index("s")
        wid = s * _NC + c

        pltpu.sync_copy(zeros_hbm.at[pl.ds(s * npt, npt)],
                        acc_v.at[pl.ds(s * npt, npt)])
        if rem:
            @pl.when(s == _NS - 1)
            def _():
                pltpu.sync_copy(zeros_hbm.at[pl.ds(_NS * npt, rem)],
                                acc_v.at[pl.ds(_NS * npt, rem)])
        plsc.subcore_barrier()

        def block(k, carry):
            pltpu.sync_copy(gidx_hbm.at[wid, k], idx_v)
            pltpu.sync_copy(dst_hbm.at[wid, k], dst_v)
            pltpu.async_copy(hw_hbm.at[idx_v.at[0]], rows_a, sem_a)

            def body(i, c2):
                j = 2 * i
                pltpu.async_copy(hw_hbm.at[idx_v.at[j + 1]], rows_b, sem_b)
                pltpu.make_async_copy(hw_hbm.at[idx_v.at[j]], rows_a,
                                      sem_a).wait()
                pltpu.sync_copy(rows_a, acc_v.at[dst_v.at[j]], add=True)

                @pl.when(j + 2 < cpb)
                def _():
                    pltpu.async_copy(hw_hbm.at[idx_v.at[j + 2]], rows_a, sem_a)

                pltpu.make_async_copy(hw_hbm.at[idx_v.at[j + 1]], rows_b,
                                      sem_b).wait()
                pltpu.sync_copy(rows_b, acc_v.at[dst_v.at[j + 1]], add=True)
                return c2

            lax.fori_loop(0, cpb // 2, body, 0)
            if cpb % 2:
                j_last = cpb - 1
                pltpu.make_async_copy(hw_hbm.at[idx_v.at[j_last]], rows_a,
                                      sem_a).wait()
                pltpu.sync_copy(rows_a, acc_v.at[dst_v.at[j_last]], add=True)
            return carry

        lax.fori_loop(0, nblk, block, 0)

        plsc.subcore_barrier()
        pltpu.sync_copy(acc_v.at[pl.ds(s * npt, npt)],
                        out_hbm.at[c, pl.ds(s * npt, npt)])
        if rem:
            @pl.when(s == _NS - 1)
            def _():
                pltpu.sync_copy(acc_v.at[pl.ds(_NS * npt, rem)],
                                out_hbm.at[c, pl.ds(_NS * npt, rem)])

    return edge_agg


def _layer(h, gidx3, dst3, zeros_nd, wb, wc, loop_w, bias2d, edge_agg, nbk):
    w = _compose_w(wc, wb)
    hw = _transform(h, w, nbk)
    part = edge_agg(hw, gidx3, dst3, zeros_nd)
    return _combine(part, h, loop_w, bias2d, nbk)


def kernel(features, edge_index, etypes, Wb1, wc1, loopW1, b1,
           Wb2, wc2, loopW2, b2):
    n, d = features.shape
    e = etypes.shape[0]
    src = edge_index[0]
    dst = edge_index[1]

    gidx = _flat_gather_index(etypes, src, n)
    epw = e // _NW
    nchunk = epw // _CHUNK
    nblk = 5
    gidx3 = gidx.reshape(_NW, nblk, nchunk // nblk, _CHUNK)
    dst3 = dst.reshape(_NW, nblk, nchunk // nblk, _CHUNK)
    zeros_nd = jnp.zeros((n, d), jnp.float32)

    edge_agg = _make_edge_agg(n, d, e)
    nbk = 5000

    h1 = _layer(features, gidx3, dst3, zeros_nd, Wb1, wc1, loopW1,
                b1.reshape(1, d), edge_agg, nbk)
    h2 = _layer(h1, gidx3, dst3, zeros_nd, Wb2, wc2, loopW2,
                b2.reshape(1, d), edge_agg, nbk)
    return h2
